# baseline (speedup 1.0000x reference)
import functools

import jax
import jax.numpy as jnp
from jax import lax
from jax.experimental import pallas as pl
from jax.experimental.pallas import tpu as pltpu
from jax.experimental.pallas import tpu_sc as plsc

VOCAB = 1000000
D_MODEL = 64
BATCH = 4096
SEQ = 200

_NW = 32
_RPW = BATCH // _NW


@functools.cache
def _build_sc_gather():
    mesh = plsc.VectorSubcoreMesh(core_axis_name="c", subcore_axis_name="s")

    @functools.partial(
        pl.kernel,
        mesh=mesh,
        compiler_params=pltpu.CompilerParams(use_tc_tiling_on_sc=False),
        out_type=jax.ShapeDtypeStruct((BATCH, SEQ, D_MODEL), jnp.float32),
        scratch_types=[
            pltpu.VMEM((_RPW, SEQ), jnp.int32),
            pltpu.VMEM((SEQ, D_MODEL), jnp.float32),
            pltpu.VMEM((SEQ, D_MODEL), jnp.float32),
            pltpu.SemaphoreType.DMA,
            pltpu.SemaphoreType.DMA,
        ],
    )
    def _sc_gather(x_hbm, table_hbm, out_hbm, idx_v, buf0, buf1, sem0, sem1):
        wid = lax.axis_index("s") * 2 + lax.axis_index("c")
        row_base = wid * _RPW
        pltpu.sync_copy(x_hbm.at[pl.ds(row_base, _RPW)], idx_v)

        def gather(r, buf, sem):
            return pltpu.async_copy(table_hbm.at[idx_v.at[r]], buf, sem)

        def gwait(buf, sem):
            pltpu.make_async_copy(table_hbm.at[idx_v.at[0]], buf, sem).wait()

        def put(r, buf):
            pltpu.sync_copy(buf, out_hbm.at[row_base + r])

        gather(0, buf0, sem0)

        def body(i, carry):
            r = 2 * i
            gather(r + 1, buf1, sem1)
            gwait(buf0, sem0)
            put(r, buf0)
            gather(r + 2, buf0, sem0)
            gwait(buf1, sem1)
            put(r + 1, buf1)
            return carry

        lax.fori_loop(0, _RPW // 2 - 1, body, 0)

        r = _RPW - 2
        gather(r + 1, buf1, sem1)
        gwait(buf0, sem0)
        put(r, buf0)
        gwait(buf1, sem1)
        put(r + 1, buf1)

    return _sc_gather


def kernel(x, table):
    return _build_sc_gather()(x, table)
